# baseline (speedup 1.0000x reference)
import jax
import jax.numpy as jnp
from jax.experimental import pallas as pl
from jax.experimental.pallas import tpu as pltpu
from jax.experimental.pallas import tpu_sc as plsc

E = 64
K = 2
D = 1024
H = 1024
T = 4096
C = 128
TRASH = E * C
BUF_ROWS = E * C + C
SUB = 8
SW = 128
SC_WIN = 256


def _router_body(x_ref, wg_ref, dsts_ref, dstg_ref, w_ref, counts_ref, laux_ref,
                 x8_ref):
    x = x_ref[...]
    wg = wg_ref[...]
    for q in range(SUB):
        x8_ref[pl.Slice(q, T, SUB), :] = x[:, q * SW:(q + 1) * SW]
    logits = jnp.dot(x, wg, preferred_element_type=jnp.float32)
    iota_e = jax.lax.broadcasted_iota(jnp.int32, (T, E), 1)
    m1 = jnp.max(logits, axis=1, keepdims=True)
    i1 = jnp.min(jnp.where(logits == m1, iota_e, E), axis=1)[:, None]
    oh1 = iota_e == i1
    masked = jnp.where(oh1, -jnp.inf, logits)
    m2 = jnp.max(masked, axis=1, keepdims=True)
    i2 = jnp.min(jnp.where(masked == m2, iota_e, E), axis=1)[:, None]
    oh2 = iota_e == i2
    probs = jax.nn.softmax(logits, axis=-1)
    p1 = jnp.sum(jnp.where(oh1, probs, 0.0), axis=1, keepdims=True)
    p2 = jnp.sum(jnp.where(oh2, probs, 0.0), axis=1, keepdims=True)
    den = p1 + p2
    g1 = p1 / den
    g2 = p2 / den
    me_sum = jnp.sum(probs, axis=0, keepdims=True)
    c1 = jnp.sum(oh1.astype(jnp.float32), axis=0, keepdims=True)
    laux_ref[...] = (E / (T * T)) * jnp.sum(me_sum * c1, axis=1, keepdims=True)
    B = oh1.astype(jnp.int32) + oh2.astype(jnp.int32)
    NB = 32
    BL = T // NB
    B3 = B.reshape(NB, BL, E)
    blk = jnp.sum(B3, axis=1)
    inc3 = B3
    s = 1
    while s < BL:
        inc3 = inc3 + jnp.concatenate(
            [jnp.zeros((NB, s, E), jnp.int32), inc3[:, : BL - s]], axis=1)
        s *= 2
    binc = blk
    s = 1
    while s < NB:
        binc = binc + jnp.concatenate(
            [jnp.zeros((s, E), jnp.int32), binc[: NB - s]], axis=0)
        s *= 2
    bpre = binc - blk
    inc = (inc3 + bpre[:, None, :]).reshape(T, E)
    pre = inc - B
    counts_ref[...] = jnp.sum(B, axis=0, keepdims=True)
    pos1 = jnp.sum(jnp.where(oh1, pre, 0), axis=1, keepdims=True)
    pos2 = jnp.sum(jnp.where(oh2, pre, 0), axis=1, keepdims=True)
    in1 = pos1 < C
    in2 = pos2 < C
    dsts1 = jnp.where(in1, i1 * C + pos1, TRASH + (pos1 % C))
    dsts2 = jnp.where(in2, i2 * C + pos2, TRASH + (pos2 % C))
    dstg1 = i1 * C + jnp.minimum(pos1, C - 1)
    dstg2 = i2 * C + jnp.minimum(pos2, C - 1)
    dsts_ref[...] = jnp.concatenate([dsts1, dsts2], axis=1)
    dstg_ref[...] = jnp.concatenate([dstg1, dstg2], axis=1)
    w_ref[...] = jnp.concatenate(
        [g1 * in1.astype(jnp.float32), g2 * in2.astype(jnp.float32)], axis=1)


def _router(x, Wg):
    return pl.pallas_call(
        _router_body,
        out_shape=(
            jax.ShapeDtypeStruct((T, K), jnp.int32),
            jax.ShapeDtypeStruct((T, K), jnp.int32),
            jax.ShapeDtypeStruct((T, K), jnp.float32),
            jax.ShapeDtypeStruct((1, E), jnp.int32),
            jax.ShapeDtypeStruct((1, 1), jnp.float32),
            jax.ShapeDtypeStruct((T * SUB, SW), jnp.float32),
        ),
    )(x, Wg)


def _sc_mesh():
    return plsc.VectorSubcoreMesh(core_axis_name="core", subcore_axis_name="subcore")


def _dispatch(x8, dsts8):
    nsub = T * SUB // SC_WIN

    @pl.kernel(
        out_type=jax.ShapeDtypeStruct((BUF_ROWS * SUB, SW), jnp.float32),
        mesh=_sc_mesh(),
    )
    def k(x_hbm, i_hbm, buf_hbm):
        def body(x_vmem, i_vmem):
            pltpu.sync_copy(x_vmem, buf_hbm.at[i_vmem.at[0]])

        pltpu.emit_pipeline(
            body,
            grid=(K * T * SUB // SC_WIN,),
            in_specs=[
                pl.BlockSpec((SC_WIN, SW), index_map=lambda i: (i % nsub, 0)),
                pl.BlockSpec((1, SC_WIN), index_map=lambda i: (0, i)),
            ],
            out_specs=[],
            core_axis_name=("core", "subcore"),
            dimension_semantics=(pltpu.PARALLEL,),
        )(x_hbm, i_hbm)

    return k(x8, dsts8)


def _sc_gather(obuf8, dstg8):
    @pl.kernel(
        out_type=jax.ShapeDtypeStruct((K * T * SUB, SW), jnp.float32),
        mesh=_sc_mesh(),
    )
    def k(obuf_hbm, i_hbm, y_hbm):
        def body(i_vmem, y_vmem):
            pltpu.sync_copy(obuf_hbm.at[i_vmem.at[0]], y_vmem)

        pltpu.emit_pipeline(
            body,
            grid=(K * T * SUB // SC_WIN,),
            in_specs=[pl.BlockSpec((1, SC_WIN), index_map=lambda i: (0, i))],
            out_specs=[pl.BlockSpec((SC_WIN, SW), index_map=lambda i: (i, 0))],
            core_axis_name=("core", "subcore"),
            dimension_semantics=(pltpu.PARALLEL,),
        )(i_hbm, y_hbm)

    return k(obuf8, dstg8)


EPB = 1


H2 = H // 2


def _ffn_body(counts_ref, buf_ref, w1_ref, w3_ref, w2_ref, obuf_ref):
    e = pl.program_id(0)
    h = pl.program_id(1)
    rid = jax.lax.broadcasted_iota(jnp.int32, (C, 1), 0)
    n = jnp.minimum(counts_ref[e], C)
    xb = jnp.concatenate(
        [buf_ref[pl.Slice(q, C, SUB), :] for q in range(SUB)], axis=1)
    xb = jnp.where(rid < n, xb, 0.0)
    h1 = jnp.dot(xb, w1_ref[0], preferred_element_type=jnp.float32)
    h3 = jnp.dot(xb, w3_ref[0], preferred_element_type=jnp.float32)
    act = jax.nn.silu(h1) * h3
    ob = jnp.dot(act, w2_ref[0], preferred_element_type=jnp.float32)

    @pl.when(h == 0)
    def _():
        for q in range(SUB):
            obuf_ref[pl.Slice(q, C, SUB), :] = ob[:, q * SW:(q + 1) * SW]

    @pl.when(h == 1)
    def _():
        for q in range(SUB):
            obuf_ref[pl.Slice(q, C, SUB), :] += ob[:, q * SW:(q + 1) * SW]


def _ffn(counts, buf8, W1, W3, W2):
    grid_spec = pltpu.PrefetchScalarGridSpec(
        num_scalar_prefetch=1,
        grid=(E, 2),
        in_specs=[
            pl.BlockSpec((C * SUB, SW), lambda e, h, s: (e, 0)),
            pl.BlockSpec((1, D, H2), lambda e, h, s: (e, 0, h)),
            pl.BlockSpec((1, D, H2), lambda e, h, s: (e, 0, h)),
            pl.BlockSpec((1, H2, D), lambda e, h, s: (e, h, 0)),
        ],
        out_specs=pl.BlockSpec((C * SUB, SW), lambda e, h, s: (e, 0)),
    )
    return pl.pallas_call(
        _ffn_body,
        grid_spec=grid_spec,
        out_shape=jax.ShapeDtypeStruct((E * C * SUB, SW), jnp.float32),
        compiler_params=pltpu.CompilerParams(
            dimension_semantics=("arbitrary", "arbitrary")),
    )(counts, buf8, W1, W3, W2)


CB = 512


def _add_body(y0_ref, y1_ref, w_ref, out_ref):
    w = w_ref[...]
    w0 = w[:, 0:1]
    w1 = w[:, 1:2]
    for q in range(SUB):
        out_ref[:, q * SW:(q + 1) * SW] = (
            y0_ref[pl.Slice(q, CB, SUB), :] * w0
            + y1_ref[pl.Slice(q, CB, SUB), :] * w1)


def _combine_add(y8, w):
    nb = T // CB
    return pl.pallas_call(
        _add_body,
        grid=(nb,),
        in_specs=[
            pl.BlockSpec((CB * SUB, SW), lambda i: (i, 0)),
            pl.BlockSpec((CB * SUB, SW), lambda i: (i + nb, 0)),
            pl.BlockSpec((CB, K), lambda i: (i, 0)),
        ],
        out_specs=pl.BlockSpec((CB, D), lambda i: (i, 0)),
        out_shape=jax.ShapeDtypeStruct((T, D), jnp.float32),
    )(y8, y8, w)


def kernel(hidden_states, Wg, W1, W3, W2):
    B, S, d = hidden_states.shape
    x = hidden_states.reshape(T, d)
    dsts, dstg, w, counts, laux, x8 = _router(x, Wg)
    counts = counts.reshape(E)
    sub = jnp.arange(SUB, dtype=jnp.int32)
    dsts8 = (dsts.T[:, :, None] * SUB + sub).reshape(1, K * T * SUB)
    dstg8 = (dstg.T[:, :, None] * SUB + sub).reshape(1, K * T * SUB)
    buf8 = _dispatch(x8, dsts8)
    obuf8 = _ffn(counts, buf8, W1, W3, W2)
    y8 = _sc_gather(obuf8, dstg8)
    out = _combine_add(y8, w)
    return out.reshape(B, S, d), laux.reshape(()), counts


# speedup vs baseline: 1.0012x; 1.0012x over previous
import jax
import jax.numpy as jnp
from jax.experimental import pallas as pl
from jax.experimental.pallas import tpu as pltpu
from jax.experimental.pallas import tpu_sc as plsc

E = 64
K = 2
D = 1024
H = 1024
T = 4096
C = 128
TRASH = E * C
BUF_ROWS = E * C + C
SUB = 8
SW = 128
SC_WIN = 256


def _router_body(x_ref, wg_ref, dsts_ref, dstg_ref, w_ref, counts_ref, laux_ref,
                 x8_ref):
    x = x_ref[...]
    wg = wg_ref[...]
    for q in range(SUB):
        x8_ref[pl.Slice(q, T, SUB), :] = x[:, q * SW:(q + 1) * SW]
    logits = jnp.dot(x, wg, preferred_element_type=jnp.float32)
    iota_e = jax.lax.broadcasted_iota(jnp.int32, (T, E), 1)
    m1 = jnp.max(logits, axis=1, keepdims=True)
    i1 = jnp.min(jnp.where(logits == m1, iota_e, E), axis=1)[:, None]
    oh1 = iota_e == i1
    masked = jnp.where(oh1, -jnp.inf, logits)
    m2 = jnp.max(masked, axis=1, keepdims=True)
    i2 = jnp.min(jnp.where(masked == m2, iota_e, E), axis=1)[:, None]
    oh2 = iota_e == i2
    probs = jax.nn.softmax(logits, axis=-1)
    p1 = jnp.sum(jnp.where(oh1, probs, 0.0), axis=1, keepdims=True)
    p2 = jnp.sum(jnp.where(oh2, probs, 0.0), axis=1, keepdims=True)
    den = p1 + p2
    g1 = p1 / den
    g2 = p2 / den
    me_sum = jnp.sum(probs, axis=0, keepdims=True)
    c1 = jnp.sum(oh1.astype(jnp.float32), axis=0, keepdims=True)
    laux_ref[...] = (E / (T * T)) * jnp.sum(me_sum * c1, axis=1, keepdims=True)
    B = oh1.astype(jnp.int32) + oh2.astype(jnp.int32)
    NB = 32
    BL = T // NB
    B3 = B.reshape(NB, BL, E)
    blk = jnp.sum(B3, axis=1)
    inc3 = B3
    s = 1
    while s < BL:
        inc3 = inc3 + jnp.concatenate(
            [jnp.zeros((NB, s, E), jnp.int32), inc3[:, : BL - s]], axis=1)
        s *= 2
    binc = blk
    s = 1
    while s < NB:
        binc = binc + jnp.concatenate(
            [jnp.zeros((s, E), jnp.int32), binc[: NB - s]], axis=0)
        s *= 2
    bpre = binc - blk
    inc = (inc3 + bpre[:, None, :]).reshape(T, E)
    pre = inc - B
    counts_ref[...] = jnp.sum(B, axis=0, keepdims=True)
    pos1 = jnp.sum(jnp.where(oh1, pre, 0), axis=1, keepdims=True)
    pos2 = jnp.sum(jnp.where(oh2, pre, 0), axis=1, keepdims=True)
    in1 = pos1 < C
    in2 = pos2 < C
    dsts1 = jnp.where(in1, i1 * C + pos1, TRASH + (pos1 % C))
    dsts2 = jnp.where(in2, i2 * C + pos2, TRASH + (pos2 % C))
    dstg1 = i1 * C + jnp.minimum(pos1, C - 1)
    dstg2 = i2 * C + jnp.minimum(pos2, C - 1)
    dsts_ref[...] = jnp.concatenate([dsts1, dsts2], axis=1)
    dstg_ref[...] = jnp.concatenate([dstg1, dstg2], axis=1)
    w_ref[...] = jnp.concatenate(
        [g1 * in1.astype(jnp.float32), g2 * in2.astype(jnp.float32)], axis=1)


def _router(x, Wg):
    return pl.pallas_call(
        _router_body,
        out_shape=(
            jax.ShapeDtypeStruct((T, K), jnp.int32),
            jax.ShapeDtypeStruct((T, K), jnp.int32),
            jax.ShapeDtypeStruct((T, K), jnp.float32),
            jax.ShapeDtypeStruct((1, E), jnp.int32),
            jax.ShapeDtypeStruct((1, 1), jnp.float32),
            jax.ShapeDtypeStruct((T * SUB, SW), jnp.float32),
        ),
    )(x, Wg)


def _sc_mesh():
    return plsc.VectorSubcoreMesh(core_axis_name="core", subcore_axis_name="subcore")


def _dispatch(x8, dsts8):
    nsub = T * SUB // SC_WIN

    @pl.kernel(
        out_type=jax.ShapeDtypeStruct((BUF_ROWS * SUB, SW), jnp.float32),
        mesh=_sc_mesh(),
    )
    def k(x_hbm, i_hbm, buf_hbm):
        def body(x_vmem, i_vmem):
            pltpu.sync_copy(x_vmem, buf_hbm.at[i_vmem.at[0]])

        pltpu.emit_pipeline(
            body,
            grid=(K * T * SUB // SC_WIN,),
            in_specs=[
                pl.BlockSpec((SC_WIN, SW), index_map=lambda i: (i % nsub, 0)),
                pl.BlockSpec((1, SC_WIN), index_map=lambda i: (0, i)),
            ],
            out_specs=[],
            core_axis_name=("core", "subcore"),
            dimension_semantics=(pltpu.PARALLEL,),
        )(x_hbm, i_hbm)

    return k(x8, dsts8)


def _sc_gather(obuf8, dstg8):
    @pl.kernel(
        out_type=jax.ShapeDtypeStruct((K * T * SUB, SW), jnp.float32),
        mesh=_sc_mesh(),
    )
    def k(obuf_hbm, i_hbm, y_hbm):
        def body(i_vmem, y_vmem):
            pltpu.sync_copy(obuf_hbm.at[i_vmem.at[0]], y_vmem)

        pltpu.emit_pipeline(
            body,
            grid=(K * T * SUB // SC_WIN,),
            in_specs=[pl.BlockSpec((1, SC_WIN), index_map=lambda i: (0, i))],
            out_specs=[pl.BlockSpec((SC_WIN, SW), index_map=lambda i: (i, 0))],
            core_axis_name=("core", "subcore"),
            dimension_semantics=(pltpu.PARALLEL,),
        )(i_hbm, y_hbm)

    return k(obuf8, dstg8)


EPB = 1


def _ffn_body(counts_ref, buf_ref, w1_ref, w3_ref, w2_ref, obuf_ref):
    e = pl.program_id(0)
    rid = jax.lax.broadcasted_iota(jnp.int32, (C, 1), 0)
    for ee in range(EPB):
        n = jnp.minimum(counts_ref[e * EPB + ee], C)
        base = ee * C * SUB
        xb = jnp.concatenate(
            [buf_ref[pl.Slice(base + q, C, SUB), :] for q in range(SUB)], axis=1)
        xb = jnp.where(rid < n, xb, 0.0)
        h1 = jnp.dot(xb, w1_ref[ee], preferred_element_type=jnp.float32)
        h3 = jnp.dot(xb, w3_ref[ee], preferred_element_type=jnp.float32)
        act = jax.nn.silu(h1) * h3
        ob = jnp.dot(act, w2_ref[ee], preferred_element_type=jnp.float32)
        for q in range(SUB):
            obuf_ref[pl.Slice(base + q, C, SUB), :] = ob[:, q * SW:(q + 1) * SW]


def _ffn(counts, buf8, W1, W3, W2):
    grid_spec = pltpu.PrefetchScalarGridSpec(
        num_scalar_prefetch=1,
        grid=(E // EPB,),
        in_specs=[
            pl.BlockSpec((EPB * C * SUB, SW), lambda e, s: (e, 0)),
            pl.BlockSpec((EPB, D, H), lambda e, s: (e, 0, 0)),
            pl.BlockSpec((EPB, D, H), lambda e, s: (e, 0, 0)),
            pl.BlockSpec((EPB, H, D), lambda e, s: (e, 0, 0)),
        ],
        out_specs=pl.BlockSpec((EPB * C * SUB, SW), lambda e, s: (e, 0)),
    )
    return pl.pallas_call(
        _ffn_body,
        grid_spec=grid_spec,
        out_shape=jax.ShapeDtypeStruct((E * C * SUB, SW), jnp.float32),
        compiler_params=pltpu.CompilerParams(
            dimension_semantics=("arbitrary",)),
    )(counts, buf8, W1, W3, W2)


CB = 512


def _add_body(y0_ref, y1_ref, w_ref, out_ref):
    w = w_ref[...]
    w0 = w[:, 0:1]
    w1 = w[:, 1:2]
    for q in range(SUB):
        out_ref[:, q * SW:(q + 1) * SW] = (
            y0_ref[pl.Slice(q, CB, SUB), :] * w0
            + y1_ref[pl.Slice(q, CB, SUB), :] * w1)


def _combine_add(y8, w):
    nb = T // CB
    return pl.pallas_call(
        _add_body,
        grid=(nb,),
        in_specs=[
            pl.BlockSpec((CB * SUB, SW), lambda i: (i, 0)),
            pl.BlockSpec((CB * SUB, SW), lambda i: (i + nb, 0)),
            pl.BlockSpec((CB, K), lambda i: (i, 0)),
        ],
        out_specs=pl.BlockSpec((CB, D), lambda i: (i, 0)),
        out_shape=jax.ShapeDtypeStruct((T, D), jnp.float32),
    )(y8, y8, w)


def kernel(hidden_states, Wg, W1, W3, W2):
    B, S, d = hidden_states.shape
    x = hidden_states.reshape(T, d)
    dsts, dstg, w, counts, laux, x8 = _router(x, Wg)
    counts = counts.reshape(E)
    sub = jnp.arange(SUB, dtype=jnp.int32)
    dsts8 = (dsts.T[:, :, None] * SUB + sub).reshape(1, K * T * SUB)
    dstg8 = (dstg.T[:, :, None] * SUB + sub).reshape(1, K * T * SUB)
    buf8 = _dispatch(x8, dsts8)
    obuf8 = _ffn(counts, buf8, W1, W3, W2)
    y8 = _sc_gather(obuf8, dstg8)
    out = _combine_add(y8, w)
    return out.reshape(B, S, d), laux.reshape(()), counts


# speedup vs baseline: 1.0087x; 1.0074x over previous
import jax
import jax.numpy as jnp
from jax.experimental import pallas as pl
from jax.experimental.pallas import tpu as pltpu
from jax.experimental.pallas import tpu_sc as plsc

E = 64
K = 2
D = 1024
H = 1024
T = 4096
C = 128
TRASH = E * C
BUF_ROWS = E * C + C
SUB = 8
SW = 128
SC_WIN = 256


def _router_body(x_ref, wg_ref, dsts_ref, dstg_ref, w_ref, counts_ref, laux_ref,
                 x8_ref):
    x = x_ref[...]
    wg = wg_ref[...]
    for q in range(SUB):
        x8_ref[pl.Slice(q, T, SUB), :] = x[:, q * SW:(q + 1) * SW]
    logits = jnp.dot(x, wg, preferred_element_type=jnp.float32)
    iota_e = jax.lax.broadcasted_iota(jnp.int32, (T, E), 1)
    m1 = jnp.max(logits, axis=1, keepdims=True)
    i1 = jnp.min(jnp.where(logits == m1, iota_e, E), axis=1)[:, None]
    oh1 = iota_e == i1
    masked = jnp.where(oh1, -jnp.inf, logits)
    m2 = jnp.max(masked, axis=1, keepdims=True)
    i2 = jnp.min(jnp.where(masked == m2, iota_e, E), axis=1)[:, None]
    oh2 = iota_e == i2
    probs = jax.nn.softmax(logits, axis=-1)
    p1 = jnp.sum(jnp.where(oh1, probs, 0.0), axis=1, keepdims=True)
    p2 = jnp.sum(jnp.where(oh2, probs, 0.0), axis=1, keepdims=True)
    den = p1 + p2
    g1 = p1 / den
    g2 = p2 / den
    me_sum = jnp.sum(probs, axis=0, keepdims=True)
    c1 = jnp.sum(oh1.astype(jnp.float32), axis=0, keepdims=True)
    laux_ref[...] = (E / (T * T)) * jnp.sum(me_sum * c1, axis=1, keepdims=True)
    B = oh1.astype(jnp.int32) + oh2.astype(jnp.int32)
    NB = 32
    BL = T // NB
    B3 = B.reshape(NB, BL, E)
    blk = jnp.sum(B3, axis=1)
    inc3 = B3
    s = 1
    while s < BL:
        inc3 = inc3 + jnp.concatenate(
            [jnp.zeros((NB, s, E), jnp.int32), inc3[:, : BL - s]], axis=1)
        s *= 2
    binc = blk
    s = 1
    while s < NB:
        binc = binc + jnp.concatenate(
            [jnp.zeros((s, E), jnp.int32), binc[: NB - s]], axis=0)
        s *= 2
    bpre = binc - blk
    inc = (inc3 + bpre[:, None, :]).reshape(T, E)
    pre = inc - B
    counts_ref[...] = jnp.sum(B, axis=0, keepdims=True)
    pos1 = jnp.sum(jnp.where(oh1, pre, 0), axis=1, keepdims=True)
    pos2 = jnp.sum(jnp.where(oh2, pre, 0), axis=1, keepdims=True)
    in1 = pos1 < C
    in2 = pos2 < C
    dsts1 = jnp.where(in1, i1 * C + pos1, TRASH + (pos1 % C))
    dsts2 = jnp.where(in2, i2 * C + pos2, TRASH + (pos2 % C))
    dstg1 = i1 * C + jnp.minimum(pos1, C - 1)
    dstg2 = i2 * C + jnp.minimum(pos2, C - 1)
    dsts_ref[...] = jnp.concatenate([dsts1, dsts2], axis=1)
    dstg_ref[...] = jnp.concatenate([dstg1, dstg2], axis=1)
    w_ref[...] = jnp.concatenate(
        [g1 * in1.astype(jnp.float32), g2 * in2.astype(jnp.float32)], axis=1)


def _router(x, Wg):
    return pl.pallas_call(
        _router_body,
        out_shape=(
            jax.ShapeDtypeStruct((T, K), jnp.int32),
            jax.ShapeDtypeStruct((T, K), jnp.int32),
            jax.ShapeDtypeStruct((T, K), jnp.float32),
            jax.ShapeDtypeStruct((1, E), jnp.int32),
            jax.ShapeDtypeStruct((1, 1), jnp.float32),
            jax.ShapeDtypeStruct((T * SUB, SW), jnp.float32),
        ),
    )(x, Wg)


def _sc_mesh():
    return plsc.VectorSubcoreMesh(core_axis_name="core", subcore_axis_name="subcore")


def _dispatch(x8, dsts8):
    nsub = T * SUB // SC_WIN

    @pl.kernel(
        out_type=jax.ShapeDtypeStruct((BUF_ROWS * SUB, SW), jnp.float32),
        mesh=_sc_mesh(),
    )
    def k(x_hbm, i_hbm, buf_hbm):
        def body(x_vmem, i_vmem):
            pltpu.sync_copy(x_vmem, buf_hbm.at[i_vmem.at[0]])

        pltpu.emit_pipeline(
            body,
            grid=(K * T * SUB // SC_WIN,),
            in_specs=[
                pl.BlockSpec((SC_WIN, SW), index_map=lambda i: (i % nsub, 0)),
                pl.BlockSpec((1, SC_WIN), index_map=lambda i: (0, i)),
            ],
            out_specs=[],
            core_axis_name=("core", "subcore"),
            dimension_semantics=(pltpu.PARALLEL,),
        )(x_hbm, i_hbm)

    return k(x8, dsts8)


def _sc_gather(obuf8, dstg8):
    @pl.kernel(
        out_type=jax.ShapeDtypeStruct((K * T * SUB, SW), jnp.float32),
        mesh=_sc_mesh(),
    )
    def k(obuf_hbm, i_hbm, y_hbm):
        def body(i_vmem, y_vmem):
            pltpu.sync_copy(obuf_hbm.at[i_vmem.at[0]], y_vmem)

        pltpu.emit_pipeline(
            body,
            grid=(K * T * SUB // SC_WIN,),
            in_specs=[pl.BlockSpec((1, SC_WIN), index_map=lambda i: (0, i))],
            out_specs=[pl.BlockSpec((SC_WIN, SW), index_map=lambda i: (i, 0))],
            core_axis_name=("core", "subcore"),
            dimension_semantics=(pltpu.PARALLEL,),
        )(i_hbm, y_hbm)

    return k(obuf8, dstg8)


EPB = 1


def _ffn_body(counts_ref, buf_ref, w1_ref, w3_ref, w2_ref, obuf_ref):
    e = pl.program_id(0)
    rid = jax.lax.broadcasted_iota(jnp.int32, (C, 1), 0)
    for ee in range(EPB):
        n = jnp.minimum(counts_ref[e * EPB + ee], C)
        base = ee * C * SUB
        xb = jnp.concatenate(
            [buf_ref[pl.Slice(base + q, C, SUB), :] for q in range(SUB)], axis=1)
        xb = jnp.where(rid < n, xb, 0.0)
        h1 = jnp.dot(xb, w1_ref[ee], preferred_element_type=jnp.float32)
        h3 = jnp.dot(xb, w3_ref[ee], preferred_element_type=jnp.float32)
        act = jax.nn.silu(h1) * h3
        ob = jnp.dot(act, w2_ref[ee], preferred_element_type=jnp.float32)
        for q in range(SUB):
            obuf_ref[pl.Slice(base + q, C, SUB), :] = ob[:, q * SW:(q + 1) * SW]


def _ffn(counts, buf8, W1, W3, W2):
    grid_spec = pltpu.PrefetchScalarGridSpec(
        num_scalar_prefetch=1,
        grid=(E // EPB,),
        in_specs=[
            pl.BlockSpec((EPB * C * SUB, SW), lambda e, s: (e, 0)),
            pl.BlockSpec((EPB, D, H), lambda e, s: (e, 0, 0)),
            pl.BlockSpec((EPB, D, H), lambda e, s: (e, 0, 0)),
            pl.BlockSpec((EPB, H, D), lambda e, s: (e, 0, 0)),
        ],
        out_specs=pl.BlockSpec((EPB * C * SUB, SW), lambda e, s: (e, 0)),
    )
    return pl.pallas_call(
        _ffn_body,
        grid_spec=grid_spec,
        out_shape=jax.ShapeDtypeStruct((E * C * SUB, SW), jnp.float32),
        compiler_params=pltpu.CompilerParams(
            dimension_semantics=("arbitrary",)),
    )(counts, buf8, W1, W3, W2)


CB = 1024


def _add_body(y0_ref, y1_ref, w_ref, out_ref):
    w = w_ref[...]
    w0 = w[:, 0:1]
    w1 = w[:, 1:2]
    for q in range(SUB):
        out_ref[:, q * SW:(q + 1) * SW] = (
            y0_ref[pl.Slice(q, CB, SUB), :] * w0
            + y1_ref[pl.Slice(q, CB, SUB), :] * w1)


def _combine_add(y8, w):
    nb = T // CB
    return pl.pallas_call(
        _add_body,
        grid=(nb,),
        in_specs=[
            pl.BlockSpec((CB * SUB, SW), lambda i: (i, 0)),
            pl.BlockSpec((CB * SUB, SW), lambda i: (i + nb, 0)),
            pl.BlockSpec((CB, K), lambda i: (i, 0)),
        ],
        out_specs=pl.BlockSpec((CB, D), lambda i: (i, 0)),
        out_shape=jax.ShapeDtypeStruct((T, D), jnp.float32),
    )(y8, y8, w)


def kernel(hidden_states, Wg, W1, W3, W2):
    B, S, d = hidden_states.shape
    x = hidden_states.reshape(T, d)
    dsts, dstg, w, counts, laux, x8 = _router(x, Wg)
    counts = counts.reshape(E)
    sub = jnp.arange(SUB, dtype=jnp.int32)
    dsts8 = (dsts.T[:, :, None] * SUB + sub).reshape(1, K * T * SUB)
    dstg8 = (dstg.T[:, :, None] * SUB + sub).reshape(1, K * T * SUB)
    buf8 = _dispatch(x8, dsts8)
    obuf8 = _ffn(counts, buf8, W1, W3, W2)
    y8 = _sc_gather(obuf8, dstg8)
    out = _combine_add(y8, w)
    return out.reshape(B, S, d), laux.reshape(()), counts
